# baseline (speedup 1.0000x reference)
import jax
import jax.numpy as jnp
from jax import lax
from jax.experimental import pallas as pl
from jax.experimental.pallas import tpu as pltpu
from jax.experimental.pallas import tpu_sc as plsc

NUM_QUERY = 4096
NUM_NEG = 128
D = 64
L = 16
GAMMA = 12.0
NW = 32
QPW = NUM_QUERY // NW
PAIR = 2 * D
PITCH = L + 1

E_ROWS = 1000000
WIN = 128
NWIN = E_ROWS // WIN
TAIL = E_ROWS - NWIN * WIN
TAIL_WORKER = NWIN % NW
TP = 73


def _tr_body(et_hbm, out_hbm, tbuf0, tbuf1, obuf0, obuf1, ttail, obufp,
             semi0, semi1, semo0, semo1):
    cid = lax.axis_index("c")
    sid = lax.axis_index("s")
    wid = sid * 2 + cid
    nk = (NWIN - wid + NW - 1) // NW

    iota65 = lax.iota(jnp.int32, L) * TP

    rows_k = [lax.iota(jnp.int32, L) + k * L for k in range(D // L)]

    def transpose_block(src, obuf, nent):
        @pl.loop(0, nent, unroll=4)
        def _i(i):
            colv = jnp.full((L,), 0, jnp.int32) + i
            for k in range(D // L):
                obuf[pl.ds(i * D + k * L, L)] = plsc.load_gather(
                    src, [rows_k[k], colv])

    def ebase(k):
        c = wid + k * NW
        return pl.multiple_of(c * WIN, WIN)

    def issue_in(k, tbuf, sem):
        pltpu.async_copy(et_hbm.at[:, pl.ds(ebase(k), WIN)], tbuf, sem)

    @pl.when(nk > 0)
    def _():
        issue_in(0, tbuf0, semi0)

    @pl.when(nk > 1)
    def _():
        issue_in(1, tbuf1, semi1)

    def wait_in(tbuf, sem):
        pltpu.make_async_copy(et_hbm.at[:, pl.ds(0, WIN)], tbuf, sem).wait()

    def wait_out(obuf, sem):
        pltpu.make_async_copy(obuf, out_hbm.at[pl.ds(0, WIN * D)], sem).wait()

    def half(kk, tbuf, obuf, semi, semo):
        @pl.when(kk < nk)
        def _():
            wait_in(tbuf, semi)

            @pl.when(kk >= 2)
            def _():
                wait_out(obuf, semo)

            transpose_block(tbuf, obuf, WIN)
            pltpu.async_copy(obuf, out_hbm.at[pl.ds(ebase(kk) * D, WIN * D)],
                             semo)

            @pl.when(kk + 2 < nk)
            def _():
                issue_in(kk + 2, tbuf, semi)

    @pl.loop(0, NWIN // NW + 2, step=2)
    def _k(k):
        half(k, tbuf0, obuf0, semi0, semo0)
        half(k + 1, tbuf1, obuf1, semi1, semo1)

    @pl.when(nk > 0)
    def _():
        wait_out(obuf0, semo0)

    @pl.when(nk > 1)
    def _():
        wait_out(obuf1, semo1)

    @pl.when(wid == TAIL_WORKER)
    def _():
        pltpu.sync_copy(et_hbm.at[:, pl.ds(NWIN * WIN, TAIL)], ttail)
        transpose_block(ttail, obuf0, TAIL)
        pltpu.async_copy(obuf0.at[pl.ds(0, TAIL * D)],
                         out_hbm.at[pl.ds(NWIN * WIN * D, TAIL * D)], semo0)
        pltpu.make_async_copy(
            obuf0.at[pl.ds(0, TAIL * D)],
            out_hbm.at[pl.ds(0, TAIL * D)], semo0).wait()


@jax.jit
def _run_tr(et):
    mesh = plsc.VectorSubcoreMesh(core_axis_name="c", subcore_axis_name="s")
    f = pl.kernel(
        _tr_body,
        out_type=jax.ShapeDtypeStruct((E_ROWS * D,), jnp.float32),
        mesh=mesh,
        compiler_params=pltpu.CompilerParams(
            needs_layout_passes=False, use_tc_tiling_on_sc=True),
        scratch_types=[
            pltpu.VMEM((D, WIN), jnp.float32),
            pltpu.VMEM((D, WIN), jnp.float32),
            pltpu.VMEM((WIN * D,), jnp.float32),
            pltpu.VMEM((WIN * D,), jnp.float32),
            pltpu.VMEM((D, TAIL), jnp.float32),
            pltpu.VMEM((WIN * TP,), jnp.float32),
            pltpu.SemaphoreType.DMA,
            pltpu.SemaphoreType.DMA,
            pltpu.SemaphoreType.DMA,
            pltpu.SemaphoreType.DMA,
        ],
    )
    return f(et)


def _sc_body(x_hbm, emb_hbm, tgtrow_hbm, tgtoff_hbm, posrow_hbm, posoff_hbm,
             negrow_hbm, negoff_hbm,
             pos_out_hbm, neg_out_hbm,
             tgtrow_v, tgtoff_v, posrow_v, posoff_v, negrow_v, negoff_v,
             pred1, pos1, nbuf0, nbuf1, t1, tpos1, pos_out_v, neg_out_v,
             sem_a, sem_n0, sem_n1):
    cid = lax.axis_index("c")
    sid = lax.axis_index("s")
    wid = sid * 2 + cid
    base = wid * QPW

    pltpu.sync_copy(tgtrow_hbm.at[pl.ds(base, QPW)], tgtrow_v)
    pltpu.sync_copy(tgtoff_hbm.at[pl.ds(base, QPW)], tgtoff_v)
    pltpu.sync_copy(posrow_hbm.at[pl.ds(base, QPW)], posrow_v)
    pltpu.sync_copy(posoff_hbm.at[pl.ds(base, QPW)], posoff_v)
    pltpu.sync_copy(negrow_hbm.at[pl.ds(base, QPW)], negrow_v)
    pltpu.sync_copy(negoff_hbm.at[pl.ds(base, QPW)], negoff_v)

    def compact(offs_v, dst1):
        @pl.loop(0, QPW // L)
        def _cp(g):
            offv = offs_v[pl.ds(g * L, L)]
            for i in range(L):
                q = g * L + i
                off = offv[i]
                for k in range(4):
                    dst1[pl.ds(q * D + k * L, L)] = (
                        nbuf0[q, pl.ds(off + k * L, L)])

    pltpu.async_copy(x_hbm.at[tgtrow_v], nbuf0, sem_a).wait()
    compact(tgtoff_v, pred1)
    pltpu.async_copy(emb_hbm.at[posrow_v], nbuf0, sem_a).wait()
    compact(posoff_v, pos1)

    pltpu.async_copy(emb_hbm.at[negrow_v.at[0]], nbuf0, sem_n0)
    pltpu.async_copy(emb_hbm.at[negrow_v.at[1]], nbuf1, sem_n1)

    def wait_nbuf(nbuf, sem):
        pltpu.make_async_copy(emb_hbm.at[pl.ds(0, NUM_NEG)], nbuf, sem).wait()

    iota = lax.iota(jnp.int32, L)
    ip = iota * PITCH

    def lane_reduce(tref, rows):
        acc = plsc.load_gather(tref, [rows])
        for c in range(1, L):
            acc = acc + plsc.load_gather(tref, [rows + c])
        return acc

    def compute(q, nbuf):
        qd = q * D
        p0 = pred1[pl.ds(qd, L)]
        p1 = pred1[pl.ds(qd + L, L)]
        p2 = pred1[pl.ds(qd + 2 * L, L)]
        p3 = pred1[pl.ds(qd + 3 * L, L)]

        a = jnp.abs(pos1[pl.ds(qd, L)] - p0)
        a = a + jnp.abs(pos1[pl.ds(qd + L, L)] - p1)
        a = a + jnp.abs(pos1[pl.ds(qd + 2 * L, L)] - p2)
        a = a + jnp.abs(pos1[pl.ds(qd + 3 * L, L)] - p3)
        tpos1[pl.ds(q * PITCH, L)] = a

        @pl.loop(0, NUM_NEG // L)
        def _grp(g):
            offv = negoff_v[q, pl.ds(g * L, L)]
            jg = g * L
            for i in range(L):
                on = offv[i]
                b = jnp.abs(nbuf[jg + i, pl.ds(on, L)] - p0)
                b = b + jnp.abs(nbuf[jg + i, pl.ds(on + L, L)] - p1)
                b = b + jnp.abs(nbuf[jg + i, pl.ds(on + 2 * L, L)] - p2)
                b = b + jnp.abs(nbuf[jg + i, pl.ds(on + 3 * L, L)] - p3)
                t1[pl.ds((jg + i) * PITCH, L)] = b

        for g in range(NUM_NEG // L):
            neg_out_v[q, pl.ds(g * L, L)] = (
                GAMMA - lane_reduce(t1, g * L * PITCH + ip))

    @pl.loop(0, QPW, step=2)
    def _q(q):
        wait_nbuf(nbuf0, sem_n0)
        compute(q, nbuf0)

        @pl.when(q + 2 < QPW)
        def _():
            pltpu.async_copy(emb_hbm.at[negrow_v.at[q + 2]], nbuf0, sem_n0)

        wait_nbuf(nbuf1, sem_n1)
        compute(q + 1, nbuf1)

        @pl.when(q + 3 < QPW)
        def _():
            pltpu.async_copy(emb_hbm.at[negrow_v.at[q + 3]], nbuf1, sem_n1)

    for g in range(QPW // L):
        pos_out_v[pl.ds(g * L, L)] = (
            GAMMA - lane_reduce(tpos1, g * L * PITCH + ip))

    pltpu.sync_copy(pos_out_v, pos_out_hbm.at[pl.ds(base, QPW)])
    pltpu.sync_copy(neg_out_v, neg_out_hbm.at[pl.ds(base, QPW)])


@jax.jit
def _run(x2, e2, tgtrow, tgtoff, posrow, posoff, negrow, negoff):
    mesh = plsc.VectorSubcoreMesh(core_axis_name="c", subcore_axis_name="s")
    f = pl.kernel(
        _sc_body,
        out_type=(
            jax.ShapeDtypeStruct((NUM_QUERY,), jnp.float32),
            jax.ShapeDtypeStruct((NUM_QUERY, NUM_NEG), jnp.float32),
        ),
        mesh=mesh,
        compiler_params=pltpu.CompilerParams(
            needs_layout_passes=False, use_tc_tiling_on_sc=True),
        scratch_types=[
            pltpu.VMEM((QPW,), jnp.int32),
            pltpu.VMEM((QPW,), jnp.int32),
            pltpu.VMEM((QPW,), jnp.int32),
            pltpu.VMEM((QPW,), jnp.int32),
            pltpu.VMEM((QPW, NUM_NEG), jnp.int32),
            pltpu.VMEM((QPW, NUM_NEG), jnp.int32),
            pltpu.VMEM((QPW * D,), jnp.float32),
            pltpu.VMEM((QPW * D,), jnp.float32),
            pltpu.VMEM((NUM_NEG, PAIR), jnp.float32),
            pltpu.VMEM((NUM_NEG, PAIR), jnp.float32),
            pltpu.VMEM((NUM_NEG * PITCH + L,), jnp.float32),
            pltpu.VMEM((QPW * PITCH + L,), jnp.float32),
            pltpu.VMEM((QPW,), jnp.float32),
            pltpu.VMEM((QPW, NUM_NEG), jnp.float32),
            pltpu.SemaphoreType.DMA,
            pltpu.SemaphoreType.DMA,
            pltpu.SemaphoreType.DMA,
        ],
    )
    return f(x2, e2, tgtrow, tgtoff, posrow, posoff, negrow, negoff)


def kernel(x, entity_embedding, target_node_idxes, positive_samples,
           negative_samples):
    tgt = target_node_idxes.astype(jnp.int32)
    pos = positive_samples.astype(jnp.int32)
    neg = negative_samples.astype(jnp.int32)
    x2 = x.reshape(x.shape[0] // 2, PAIR)
    t1d = _run_tr(entity_embedding.T)
    e2 = t1d.reshape(E_ROWS // 2, PAIR)
    pos_logit, neg_logit = _run(
        x2, e2,
        tgt >> 1, (tgt & 1) * D,
        pos >> 1, (pos & 1) * D,
        neg >> 1, (neg & 1) * D,
    )
    return (pos_logit[:, None], neg_logit)


# speedup vs baseline: 1.4653x; 1.4653x over previous
import jax
import jax.numpy as jnp
from jax import lax
from jax.experimental import pallas as pl
from jax.experimental.pallas import tpu as pltpu
from jax.experimental.pallas import tpu_sc as plsc

NUM_QUERY = 4096
NUM_NEG = 128
D = 64
L = 16
GAMMA = 12.0
NW = 32
QPW = NUM_QUERY // NW
PAIR = 2 * D
PITCH = L + 1

E_ROWS = 1000000
WIN = 128
NWIN = E_ROWS // WIN
TAIL = E_ROWS - NWIN * WIN
TAIL_WORKER = NWIN % NW
TP = WIN + 1


def _tr_body(et_hbm, out_hbm, tbuf0, tbuf1, obuf0, obuf1, ttail, obufp,
             semi0, semi1, semo0, semo1):
    cid = lax.axis_index("c")
    sid = lax.axis_index("s")
    wid = sid * 2 + cid
    nk = (NWIN - wid + NW - 1) // NW

    iota65 = lax.iota(jnp.int32, L) * TP

    rowsP_k = [(lax.iota(jnp.int32, L) + k * L) * TP for k in range(D // L)]

    def transpose_block(src, obuf, nent):
        @pl.loop(0, D, unroll=4)
        def _d(d):
            for g in range(nent // L):
                obufp[pl.ds(d * TP + g * L, L)] = src[d, pl.ds(g * L, L)]

        @pl.loop(0, nent, unroll=4)
        def _i(i):
            for k in range(D // L):
                obuf[pl.ds(i * D + k * L, L)] = plsc.load_gather(
                    obufp, [rowsP_k[k] + i])

    def ebase(k):
        c = wid + k * NW
        return pl.multiple_of(c * WIN, WIN)

    def issue_in(k, tbuf, sem):
        pltpu.async_copy(et_hbm.at[:, pl.ds(ebase(k), WIN)], tbuf, sem)

    @pl.when(nk > 0)
    def _():
        issue_in(0, tbuf0, semi0)

    @pl.when(nk > 1)
    def _():
        issue_in(1, tbuf1, semi1)

    def wait_in(tbuf, sem):
        pltpu.make_async_copy(et_hbm.at[:, pl.ds(0, WIN)], tbuf, sem).wait()

    def wait_out(obuf, sem):
        pltpu.make_async_copy(obuf, out_hbm.at[pl.ds(0, WIN * D)], sem).wait()

    def half(kk, tbuf, obuf, semi, semo):
        @pl.when(kk < nk)
        def _():
            wait_in(tbuf, semi)

            @pl.when(kk >= 2)
            def _():
                wait_out(obuf, semo)

            transpose_block(tbuf, obuf, WIN)
            pltpu.async_copy(obuf, out_hbm.at[pl.ds(ebase(kk) * D, WIN * D)],
                             semo)

            @pl.when(kk + 2 < nk)
            def _():
                issue_in(kk + 2, tbuf, semi)

    @pl.loop(0, NWIN // NW + 2, step=2)
    def _k(k):
        half(k, tbuf0, obuf0, semi0, semo0)
        half(k + 1, tbuf1, obuf1, semi1, semo1)

    @pl.when(nk > 0)
    def _():
        wait_out(obuf0, semo0)

    @pl.when(nk > 1)
    def _():
        wait_out(obuf1, semo1)

    @pl.when(wid == TAIL_WORKER)
    def _():
        pltpu.sync_copy(et_hbm.at[:, pl.ds(NWIN * WIN, TAIL)], ttail)
        transpose_block(ttail, obuf0, TAIL)
        pltpu.async_copy(obuf0.at[pl.ds(0, TAIL * D)],
                         out_hbm.at[pl.ds(NWIN * WIN * D, TAIL * D)], semo0)
        pltpu.make_async_copy(
            obuf0.at[pl.ds(0, TAIL * D)],
            out_hbm.at[pl.ds(0, TAIL * D)], semo0).wait()


@jax.jit
def _run_tr(et):
    mesh = plsc.VectorSubcoreMesh(core_axis_name="c", subcore_axis_name="s")
    f = pl.kernel(
        _tr_body,
        out_type=jax.ShapeDtypeStruct((E_ROWS * D,), jnp.float32),
        mesh=mesh,
        compiler_params=pltpu.CompilerParams(
            needs_layout_passes=False, use_tc_tiling_on_sc=True),
        scratch_types=[
            pltpu.VMEM((D, WIN), jnp.float32),
            pltpu.VMEM((D, WIN), jnp.float32),
            pltpu.VMEM((WIN * D,), jnp.float32),
            pltpu.VMEM((WIN * D,), jnp.float32),
            pltpu.VMEM((D, TAIL), jnp.float32),
            pltpu.VMEM((D * TP,), jnp.float32),
            pltpu.SemaphoreType.DMA,
            pltpu.SemaphoreType.DMA,
            pltpu.SemaphoreType.DMA,
            pltpu.SemaphoreType.DMA,
        ],
    )
    return f(et)


def _sc_body(x_hbm, emb_hbm, tgtrow_hbm, tgtoff_hbm, posrow_hbm, posoff_hbm,
             negrow_hbm, negoff_hbm,
             pos_out_hbm, neg_out_hbm,
             tgtrow_v, tgtoff_v, posrow_v, posoff_v, negrow_v, negoff_v,
             pred1, pos1, nbuf0, nbuf1, t1, tpos1, pos_out_v, neg_out_v,
             sem_a, sem_n0, sem_n1):
    cid = lax.axis_index("c")
    sid = lax.axis_index("s")
    wid = sid * 2 + cid
    base = wid * QPW

    pltpu.sync_copy(tgtrow_hbm.at[pl.ds(base, QPW)], tgtrow_v)
    pltpu.sync_copy(tgtoff_hbm.at[pl.ds(base, QPW)], tgtoff_v)
    pltpu.sync_copy(posrow_hbm.at[pl.ds(base, QPW)], posrow_v)
    pltpu.sync_copy(posoff_hbm.at[pl.ds(base, QPW)], posoff_v)
    pltpu.sync_copy(negrow_hbm.at[pl.ds(base, QPW)], negrow_v)
    pltpu.sync_copy(negoff_hbm.at[pl.ds(base, QPW)], negoff_v)

    def compact(offs_v, dst1):
        @pl.loop(0, QPW // L)
        def _cp(g):
            offv = offs_v[pl.ds(g * L, L)]
            for i in range(L):
                q = g * L + i
                off = offv[i]
                for k in range(4):
                    dst1[pl.ds(q * D + k * L, L)] = (
                        nbuf0[q, pl.ds(off + k * L, L)])

    pltpu.async_copy(x_hbm.at[tgtrow_v], nbuf0, sem_a).wait()
    compact(tgtoff_v, pred1)
    pltpu.async_copy(emb_hbm.at[posrow_v], nbuf0, sem_a).wait()
    compact(posoff_v, pos1)

    pltpu.async_copy(emb_hbm.at[negrow_v.at[0]], nbuf0, sem_n0)
    pltpu.async_copy(emb_hbm.at[negrow_v.at[1]], nbuf1, sem_n1)

    def wait_nbuf(nbuf, sem):
        pltpu.make_async_copy(emb_hbm.at[pl.ds(0, NUM_NEG)], nbuf, sem).wait()

    iota = lax.iota(jnp.int32, L)
    ip = iota * PITCH

    def lane_reduce(tref, rows):
        acc = plsc.load_gather(tref, [rows])
        for c in range(1, L):
            acc = acc + plsc.load_gather(tref, [rows + c])
        return acc

    def compute(q, nbuf):
        qd = q * D
        p0 = pred1[pl.ds(qd, L)]
        p1 = pred1[pl.ds(qd + L, L)]
        p2 = pred1[pl.ds(qd + 2 * L, L)]
        p3 = pred1[pl.ds(qd + 3 * L, L)]

        a = jnp.abs(pos1[pl.ds(qd, L)] - p0)
        a = a + jnp.abs(pos1[pl.ds(qd + L, L)] - p1)
        a = a + jnp.abs(pos1[pl.ds(qd + 2 * L, L)] - p2)
        a = a + jnp.abs(pos1[pl.ds(qd + 3 * L, L)] - p3)
        tpos1[pl.ds(q * PITCH, L)] = a

        @pl.loop(0, NUM_NEG // L)
        def _grp(g):
            offv = negoff_v[q, pl.ds(g * L, L)]
            jg = g * L
            for i in range(L):
                on = offv[i]
                b = jnp.abs(nbuf[jg + i, pl.ds(on, L)] - p0)
                b = b + jnp.abs(nbuf[jg + i, pl.ds(on + L, L)] - p1)
                b = b + jnp.abs(nbuf[jg + i, pl.ds(on + 2 * L, L)] - p2)
                b = b + jnp.abs(nbuf[jg + i, pl.ds(on + 3 * L, L)] - p3)
                t1[pl.ds((jg + i) * PITCH, L)] = b

        for g in range(NUM_NEG // L):
            neg_out_v[q, pl.ds(g * L, L)] = (
                GAMMA - lane_reduce(t1, g * L * PITCH + ip))

    @pl.loop(0, QPW, step=2)
    def _q(q):
        wait_nbuf(nbuf0, sem_n0)
        compute(q, nbuf0)

        @pl.when(q + 2 < QPW)
        def _():
            pltpu.async_copy(emb_hbm.at[negrow_v.at[q + 2]], nbuf0, sem_n0)

        wait_nbuf(nbuf1, sem_n1)
        compute(q + 1, nbuf1)

        @pl.when(q + 3 < QPW)
        def _():
            pltpu.async_copy(emb_hbm.at[negrow_v.at[q + 3]], nbuf1, sem_n1)

    for g in range(QPW // L):
        pos_out_v[pl.ds(g * L, L)] = (
            GAMMA - lane_reduce(tpos1, g * L * PITCH + ip))

    pltpu.sync_copy(pos_out_v, pos_out_hbm.at[pl.ds(base, QPW)])
    pltpu.sync_copy(neg_out_v, neg_out_hbm.at[pl.ds(base, QPW)])


@jax.jit
def _run(x2, e2, tgtrow, tgtoff, posrow, posoff, negrow, negoff):
    mesh = plsc.VectorSubcoreMesh(core_axis_name="c", subcore_axis_name="s")
    f = pl.kernel(
        _sc_body,
        out_type=(
            jax.ShapeDtypeStruct((NUM_QUERY,), jnp.float32),
            jax.ShapeDtypeStruct((NUM_QUERY, NUM_NEG), jnp.float32),
        ),
        mesh=mesh,
        compiler_params=pltpu.CompilerParams(
            needs_layout_passes=False, use_tc_tiling_on_sc=True),
        scratch_types=[
            pltpu.VMEM((QPW,), jnp.int32),
            pltpu.VMEM((QPW,), jnp.int32),
            pltpu.VMEM((QPW,), jnp.int32),
            pltpu.VMEM((QPW,), jnp.int32),
            pltpu.VMEM((QPW, NUM_NEG), jnp.int32),
            pltpu.VMEM((QPW, NUM_NEG), jnp.int32),
            pltpu.VMEM((QPW * D,), jnp.float32),
            pltpu.VMEM((QPW * D,), jnp.float32),
            pltpu.VMEM((NUM_NEG, PAIR), jnp.float32),
            pltpu.VMEM((NUM_NEG, PAIR), jnp.float32),
            pltpu.VMEM((NUM_NEG * PITCH + L,), jnp.float32),
            pltpu.VMEM((QPW * PITCH + L,), jnp.float32),
            pltpu.VMEM((QPW,), jnp.float32),
            pltpu.VMEM((QPW, NUM_NEG), jnp.float32),
            pltpu.SemaphoreType.DMA,
            pltpu.SemaphoreType.DMA,
            pltpu.SemaphoreType.DMA,
        ],
    )
    return f(x2, e2, tgtrow, tgtoff, posrow, posoff, negrow, negoff)


def kernel(x, entity_embedding, target_node_idxes, positive_samples,
           negative_samples):
    tgt = target_node_idxes.astype(jnp.int32)
    pos = positive_samples.astype(jnp.int32)
    neg = negative_samples.astype(jnp.int32)
    x2 = x.reshape(x.shape[0] // 2, PAIR)
    t1d = _run_tr(entity_embedding.T)
    e2 = t1d.reshape(E_ROWS // 2, PAIR)
    pos_logit, neg_logit = _run(
        x2, e2,
        tgt >> 1, (tgt & 1) * D,
        pos >> 1, (pos & 1) * D,
        neg >> 1, (neg & 1) * D,
    )
    return (pos_logit[:, None], neg_logit)


# speedup vs baseline: 3.5837x; 2.4457x over previous
import jax
import jax.numpy as jnp
from jax import lax
from jax.experimental import pallas as pl
from jax.experimental.pallas import tpu as pltpu
from jax.experimental.pallas import tpu_sc as plsc

NUM_QUERY = 4096
NUM_NEG = 128
D = 64
L = 16
GAMMA = 12.0
NW = 32
QPW = NUM_QUERY // NW
PAIR = 2 * D
PITCH = L + 1

E_ROWS = 1000000
WIN = 128
NWIN = E_ROWS // WIN
TAIL = E_ROWS - NWIN * WIN
TAIL_WORKER = NWIN % NW
TP = WIN + 1


def _tr_body(et_hbm, out_hbm, tbuf0, tbuf1, obuf0, obuf1, ttail, obufp,
             semi0, semi1, semo0, semo1):
    cid = lax.axis_index("c")
    sid = lax.axis_index("s")
    wid = sid * 2 + cid
    nk = (NWIN - wid + NW - 1) // NW

    iota65 = lax.iota(jnp.int32, L) * TP

    rowsP_k = [(lax.iota(jnp.int32, L) + k * L) * TP for k in range(D // L)]

    def transpose_block(src, obuf, nent):
        @plsc.parallel_loop(0, D, step=1, unroll=4)
        def _d(d):
            for g in range(nent // L):
                obufp[pl.ds(d * TP + g * L, L)] = src[d, pl.ds(g * L, L)]

        @plsc.parallel_loop(0, nent, step=1, unroll=4)
        def _i(i):
            for k in range(D // L):
                obuf[pl.ds(i * D + k * L, L)] = plsc.load_gather(
                    obufp, [rowsP_k[k] + i])

    def ebase(k):
        c = wid + k * NW
        return pl.multiple_of(c * WIN, WIN)

    def issue_in(k, tbuf, sem):
        pltpu.async_copy(et_hbm.at[:, pl.ds(ebase(k), WIN)], tbuf, sem)

    @pl.when(nk > 0)
    def _():
        issue_in(0, tbuf0, semi0)

    @pl.when(nk > 1)
    def _():
        issue_in(1, tbuf1, semi1)

    def wait_in(tbuf, sem):
        pltpu.make_async_copy(et_hbm.at[:, pl.ds(0, WIN)], tbuf, sem).wait()

    def wait_out(obuf, sem):
        pltpu.make_async_copy(obuf, out_hbm.at[pl.ds(0, WIN * D)], sem).wait()

    def half(kk, tbuf, obuf, semi, semo):
        @pl.when(kk < nk)
        def _():
            wait_in(tbuf, semi)

            @pl.when(kk >= 2)
            def _():
                wait_out(obuf, semo)

            transpose_block(tbuf, obuf, WIN)
            pltpu.async_copy(obuf, out_hbm.at[pl.ds(ebase(kk) * D, WIN * D)],
                             semo)

            @pl.when(kk + 2 < nk)
            def _():
                issue_in(kk + 2, tbuf, semi)

    @pl.loop(0, NWIN // NW + 2, step=2)
    def _k(k):
        half(k, tbuf0, obuf0, semi0, semo0)
        half(k + 1, tbuf1, obuf1, semi1, semo1)

    @pl.when(nk > 0)
    def _():
        wait_out(obuf0, semo0)

    @pl.when(nk > 1)
    def _():
        wait_out(obuf1, semo1)

    @pl.when(wid == TAIL_WORKER)
    def _():
        pltpu.sync_copy(et_hbm.at[:, pl.ds(NWIN * WIN, TAIL)], ttail)
        transpose_block(ttail, obuf0, TAIL)
        pltpu.async_copy(obuf0.at[pl.ds(0, TAIL * D)],
                         out_hbm.at[pl.ds(NWIN * WIN * D, TAIL * D)], semo0)
        pltpu.make_async_copy(
            obuf0.at[pl.ds(0, TAIL * D)],
            out_hbm.at[pl.ds(0, TAIL * D)], semo0).wait()


@jax.jit
def _run_tr(et):
    mesh = plsc.VectorSubcoreMesh(core_axis_name="c", subcore_axis_name="s")
    f = pl.kernel(
        _tr_body,
        out_type=jax.ShapeDtypeStruct((E_ROWS * D,), jnp.float32),
        mesh=mesh,
        compiler_params=pltpu.CompilerParams(
            needs_layout_passes=False, use_tc_tiling_on_sc=True),
        scratch_types=[
            pltpu.VMEM((D, WIN), jnp.float32),
            pltpu.VMEM((D, WIN), jnp.float32),
            pltpu.VMEM((WIN * D,), jnp.float32),
            pltpu.VMEM((WIN * D,), jnp.float32),
            pltpu.VMEM((D, TAIL), jnp.float32),
            pltpu.VMEM((D * TP,), jnp.float32),
            pltpu.SemaphoreType.DMA,
            pltpu.SemaphoreType.DMA,
            pltpu.SemaphoreType.DMA,
            pltpu.SemaphoreType.DMA,
        ],
    )
    return f(et)


def _sc_body(x_hbm, emb_hbm, tgtrow_hbm, tgtoff_hbm, posrow_hbm, posoff_hbm,
             negrow_hbm, negoff_hbm,
             pos_out_hbm, neg_out_hbm,
             tgtrow_v, tgtoff_v, posrow_v, posoff_v, negrow_v, negoff_v,
             pred1, pos1, nbuf0, nbuf1, t1, tpos1, pos_out_v, neg_out_v,
             sem_a, sem_n0, sem_n1):
    cid = lax.axis_index("c")
    sid = lax.axis_index("s")
    wid = sid * 2 + cid
    base = wid * QPW

    pltpu.sync_copy(tgtrow_hbm.at[pl.ds(base, QPW)], tgtrow_v)
    pltpu.sync_copy(tgtoff_hbm.at[pl.ds(base, QPW)], tgtoff_v)
    pltpu.sync_copy(posrow_hbm.at[pl.ds(base, QPW)], posrow_v)
    pltpu.sync_copy(posoff_hbm.at[pl.ds(base, QPW)], posoff_v)
    pltpu.sync_copy(negrow_hbm.at[pl.ds(base, QPW)], negrow_v)
    pltpu.sync_copy(negoff_hbm.at[pl.ds(base, QPW)], negoff_v)

    def compact(offs_v, dst1):
        @pl.loop(0, QPW // L)
        def _cp(g):
            offv = offs_v[pl.ds(g * L, L)]
            for i in range(L):
                q = g * L + i
                off = offv[i]
                for k in range(4):
                    dst1[pl.ds(q * D + k * L, L)] = (
                        nbuf0[q, pl.ds(off + k * L, L)])

    pltpu.async_copy(x_hbm.at[tgtrow_v], nbuf0, sem_a).wait()
    compact(tgtoff_v, pred1)
    pltpu.async_copy(emb_hbm.at[posrow_v], nbuf0, sem_a).wait()
    compact(posoff_v, pos1)

    pltpu.async_copy(emb_hbm.at[negrow_v.at[0]], nbuf0, sem_n0)
    pltpu.async_copy(emb_hbm.at[negrow_v.at[1]], nbuf1, sem_n1)

    def wait_nbuf(nbuf, sem):
        pltpu.make_async_copy(emb_hbm.at[pl.ds(0, NUM_NEG)], nbuf, sem).wait()

    iota = lax.iota(jnp.int32, L)
    ip = iota * PITCH

    def lane_reduce(tref, rows):
        acc = plsc.load_gather(tref, [rows])
        for c in range(1, L):
            acc = acc + plsc.load_gather(tref, [rows + c])
        return acc

    def compute(q, nbuf):
        qd = q * D
        p0 = pred1[pl.ds(qd, L)]
        p1 = pred1[pl.ds(qd + L, L)]
        p2 = pred1[pl.ds(qd + 2 * L, L)]
        p3 = pred1[pl.ds(qd + 3 * L, L)]

        a = jnp.abs(pos1[pl.ds(qd, L)] - p0)
        a = a + jnp.abs(pos1[pl.ds(qd + L, L)] - p1)
        a = a + jnp.abs(pos1[pl.ds(qd + 2 * L, L)] - p2)
        a = a + jnp.abs(pos1[pl.ds(qd + 3 * L, L)] - p3)
        tpos1[pl.ds(q * PITCH, L)] = a

        @pl.loop(0, NUM_NEG // L)
        def _grp(g):
            offv = negoff_v[q, pl.ds(g * L, L)]
            jg = g * L
            for i in range(L):
                on = offv[i]
                b = jnp.abs(nbuf[jg + i, pl.ds(on, L)] - p0)
                b = b + jnp.abs(nbuf[jg + i, pl.ds(on + L, L)] - p1)
                b = b + jnp.abs(nbuf[jg + i, pl.ds(on + 2 * L, L)] - p2)
                b = b + jnp.abs(nbuf[jg + i, pl.ds(on + 3 * L, L)] - p3)
                t1[pl.ds((jg + i) * PITCH, L)] = b

        for g in range(NUM_NEG // L):
            neg_out_v[q, pl.ds(g * L, L)] = (
                GAMMA - lane_reduce(t1, g * L * PITCH + ip))

    @pl.loop(0, QPW, step=2)
    def _q(q):
        wait_nbuf(nbuf0, sem_n0)
        compute(q, nbuf0)

        @pl.when(q + 2 < QPW)
        def _():
            pltpu.async_copy(emb_hbm.at[negrow_v.at[q + 2]], nbuf0, sem_n0)

        wait_nbuf(nbuf1, sem_n1)
        compute(q + 1, nbuf1)

        @pl.when(q + 3 < QPW)
        def _():
            pltpu.async_copy(emb_hbm.at[negrow_v.at[q + 3]], nbuf1, sem_n1)

    for g in range(QPW // L):
        pos_out_v[pl.ds(g * L, L)] = (
            GAMMA - lane_reduce(tpos1, g * L * PITCH + ip))

    pltpu.sync_copy(pos_out_v, pos_out_hbm.at[pl.ds(base, QPW)])
    pltpu.sync_copy(neg_out_v, neg_out_hbm.at[pl.ds(base, QPW)])


@jax.jit
def _run(x2, e2, tgtrow, tgtoff, posrow, posoff, negrow, negoff):
    mesh = plsc.VectorSubcoreMesh(core_axis_name="c", subcore_axis_name="s")
    f = pl.kernel(
        _sc_body,
        out_type=(
            jax.ShapeDtypeStruct((NUM_QUERY,), jnp.float32),
            jax.ShapeDtypeStruct((NUM_QUERY, NUM_NEG), jnp.float32),
        ),
        mesh=mesh,
        compiler_params=pltpu.CompilerParams(
            needs_layout_passes=False, use_tc_tiling_on_sc=True),
        scratch_types=[
            pltpu.VMEM((QPW,), jnp.int32),
            pltpu.VMEM((QPW,), jnp.int32),
            pltpu.VMEM((QPW,), jnp.int32),
            pltpu.VMEM((QPW,), jnp.int32),
            pltpu.VMEM((QPW, NUM_NEG), jnp.int32),
            pltpu.VMEM((QPW, NUM_NEG), jnp.int32),
            pltpu.VMEM((QPW * D,), jnp.float32),
            pltpu.VMEM((QPW * D,), jnp.float32),
            pltpu.VMEM((NUM_NEG, PAIR), jnp.float32),
            pltpu.VMEM((NUM_NEG, PAIR), jnp.float32),
            pltpu.VMEM((NUM_NEG * PITCH + L,), jnp.float32),
            pltpu.VMEM((QPW * PITCH + L,), jnp.float32),
            pltpu.VMEM((QPW,), jnp.float32),
            pltpu.VMEM((QPW, NUM_NEG), jnp.float32),
            pltpu.SemaphoreType.DMA,
            pltpu.SemaphoreType.DMA,
            pltpu.SemaphoreType.DMA,
        ],
    )
    return f(x2, e2, tgtrow, tgtoff, posrow, posoff, negrow, negoff)


def kernel(x, entity_embedding, target_node_idxes, positive_samples,
           negative_samples):
    tgt = target_node_idxes.astype(jnp.int32)
    pos = positive_samples.astype(jnp.int32)
    neg = negative_samples.astype(jnp.int32)
    x2 = x.reshape(x.shape[0] // 2, PAIR)
    t1d = _run_tr(entity_embedding.T)
    e2 = t1d.reshape(E_ROWS // 2, PAIR)
    pos_logit, neg_logit = _run(
        x2, e2,
        tgt >> 1, (tgt & 1) * D,
        pos >> 1, (pos & 1) * D,
        neg >> 1, (neg & 1) * D,
    )
    return (pos_logit[:, None], neg_logit)


# speedup vs baseline: 3.8805x; 1.0828x over previous
import jax
import jax.numpy as jnp
from jax import lax
from jax.experimental import pallas as pl
from jax.experimental.pallas import tpu as pltpu
from jax.experimental.pallas import tpu_sc as plsc

NUM_QUERY = 4096
NUM_NEG = 128
D = 64
L = 16
GAMMA = 12.0
NW = 32
QPW = NUM_QUERY // NW
PAIR = 2 * D
PITCH = L + 1

E_ROWS = 1000000
WIN = 128
NWIN = E_ROWS // WIN
TAIL = E_ROWS - NWIN * WIN
TAIL_WORKER = NWIN % NW
TP = WIN + 1


def _tr_body(et_hbm, out_hbm, tbuf0, tbuf1, obuf0, obuf1, ttail, obufp,
             semi0, semi1, semo0, semo1):
    cid = lax.axis_index("c")
    sid = lax.axis_index("s")
    wid = sid * 2 + cid
    nk = (NWIN - wid + NW - 1) // NW

    iota65 = lax.iota(jnp.int32, L) * TP

    rowsP_k = [(lax.iota(jnp.int32, L) + k * L) * TP for k in range(D // L)]

    def transpose_block(src, obuf, nent):
        @plsc.parallel_loop(0, D, step=1, unroll=4)
        def _d(d):
            for g in range(nent // L):
                obufp[pl.ds(d * TP + g * L, L)] = src[d, pl.ds(g * L, L)]

        @plsc.parallel_loop(0, nent, step=1, unroll=4)
        def _i(i):
            for k in range(D // L):
                obuf[pl.ds(i * D + k * L, L)] = plsc.load_gather(
                    obufp, [rowsP_k[k] + i])

    def ebase(k):
        c = wid + k * NW
        return pl.multiple_of(c * WIN, WIN)

    def issue_in(k, tbuf, sem):
        pltpu.async_copy(et_hbm.at[:, pl.ds(ebase(k), WIN)], tbuf, sem)

    @pl.when(nk > 0)
    def _():
        issue_in(0, tbuf0, semi0)

    @pl.when(nk > 1)
    def _():
        issue_in(1, tbuf1, semi1)

    def wait_in(tbuf, sem):
        pltpu.make_async_copy(et_hbm.at[:, pl.ds(0, WIN)], tbuf, sem).wait()

    def wait_out(obuf, sem):
        pltpu.make_async_copy(obuf, out_hbm.at[pl.ds(0, WIN * D)], sem).wait()

    def half(kk, tbuf, obuf, semi, semo):
        @pl.when(kk < nk)
        def _():
            wait_in(tbuf, semi)

            @pl.when(kk >= 2)
            def _():
                wait_out(obuf, semo)

            transpose_block(tbuf, obuf, WIN)
            pltpu.async_copy(obuf, out_hbm.at[pl.ds(ebase(kk) * D, WIN * D)],
                             semo)

            @pl.when(kk + 2 < nk)
            def _():
                issue_in(kk + 2, tbuf, semi)

    @pl.loop(0, NWIN // NW + 2, step=2)
    def _k(k):
        half(k, tbuf0, obuf0, semi0, semo0)
        half(k + 1, tbuf1, obuf1, semi1, semo1)

    @pl.when(nk > 0)
    def _():
        wait_out(obuf0, semo0)

    @pl.when(nk > 1)
    def _():
        wait_out(obuf1, semo1)

    @pl.when(wid == TAIL_WORKER)
    def _():
        pltpu.sync_copy(et_hbm.at[:, pl.ds(NWIN * WIN, TAIL)], ttail)
        transpose_block(ttail, obuf0, TAIL)
        pltpu.async_copy(obuf0.at[pl.ds(0, TAIL * D)],
                         out_hbm.at[pl.ds(NWIN * WIN * D, TAIL * D)], semo0)
        pltpu.make_async_copy(
            obuf0.at[pl.ds(0, TAIL * D)],
            out_hbm.at[pl.ds(0, TAIL * D)], semo0).wait()


@jax.jit
def _run_tr(et):
    mesh = plsc.VectorSubcoreMesh(core_axis_name="c", subcore_axis_name="s")
    f = pl.kernel(
        _tr_body,
        out_type=jax.ShapeDtypeStruct((E_ROWS * D,), jnp.float32),
        mesh=mesh,
        compiler_params=pltpu.CompilerParams(
            needs_layout_passes=False, use_tc_tiling_on_sc=True),
        scratch_types=[
            pltpu.VMEM((D, WIN), jnp.float32),
            pltpu.VMEM((D, WIN), jnp.float32),
            pltpu.VMEM((WIN * D,), jnp.float32),
            pltpu.VMEM((WIN * D,), jnp.float32),
            pltpu.VMEM((D, TAIL), jnp.float32),
            pltpu.VMEM((D * TP,), jnp.float32),
            pltpu.SemaphoreType.DMA,
            pltpu.SemaphoreType.DMA,
            pltpu.SemaphoreType.DMA,
            pltpu.SemaphoreType.DMA,
        ],
    )
    return f(et)


def _sc_body(x_hbm, emb_hbm, tgtrow_hbm, tgtoff_hbm, posrow_hbm, posoff_hbm,
             negrow_hbm, negoff_hbm,
             pos_out_hbm, neg_out_hbm,
             tgtrow_v, tgtoff_v, posrow_v, posoff_v, negrow_v, negoff_v,
             pred1, pos1, nbuf0, nbuf1, t1, tpos1, pos_out_v, neg_out_v,
             sem_a, sem_n0, sem_n1):
    cid = lax.axis_index("c")
    sid = lax.axis_index("s")
    wid = sid * 2 + cid
    base = wid * QPW

    pltpu.sync_copy(tgtrow_hbm.at[pl.ds(base, QPW)], tgtrow_v)
    pltpu.sync_copy(tgtoff_hbm.at[pl.ds(base, QPW)], tgtoff_v)
    pltpu.sync_copy(posrow_hbm.at[pl.ds(base, QPW)], posrow_v)
    pltpu.sync_copy(posoff_hbm.at[pl.ds(base, QPW)], posoff_v)
    pltpu.sync_copy(negrow_hbm.at[pl.ds(base, QPW)], negrow_v)
    pltpu.sync_copy(negoff_hbm.at[pl.ds(base, QPW)], negoff_v)

    def compact(offs_v, dst1):
        @plsc.parallel_loop(0, QPW // L, step=1)
        def _cp(g):
            offv = offs_v[pl.ds(g * L, L)]
            for i in range(L):
                q = g * L + i
                off = offv[i]
                for k in range(4):
                    dst1[pl.ds(q * D + k * L, L)] = (
                        nbuf0[q, pl.ds(off + k * L, L)])

    pltpu.async_copy(x_hbm.at[tgtrow_v], nbuf0, sem_a).wait()
    compact(tgtoff_v, pred1)
    pltpu.async_copy(emb_hbm.at[posrow_v], nbuf0, sem_a).wait()
    compact(posoff_v, pos1)

    pltpu.async_copy(emb_hbm.at[negrow_v.at[0]], nbuf0, sem_n0)
    pltpu.async_copy(emb_hbm.at[negrow_v.at[1]], nbuf1, sem_n1)

    def wait_nbuf(nbuf, sem):
        pltpu.make_async_copy(emb_hbm.at[pl.ds(0, NUM_NEG)], nbuf, sem).wait()

    iota = lax.iota(jnp.int32, L)
    ip = iota * PITCH

    def lane_reduce(tref, rows):
        vs = [plsc.load_gather(tref, [rows + c]) for c in range(L)]
        while len(vs) > 1:
            vs = [a + b for a, b in zip(vs[::2], vs[1::2])]
        return vs[0]

    def compute(q, nbuf):
        qd = q * D
        p0 = pred1[pl.ds(qd, L)]
        p1 = pred1[pl.ds(qd + L, L)]
        p2 = pred1[pl.ds(qd + 2 * L, L)]
        p3 = pred1[pl.ds(qd + 3 * L, L)]

        a = jnp.abs(pos1[pl.ds(qd, L)] - p0)
        a = a + jnp.abs(pos1[pl.ds(qd + L, L)] - p1)
        a = a + jnp.abs(pos1[pl.ds(qd + 2 * L, L)] - p2)
        a = a + jnp.abs(pos1[pl.ds(qd + 3 * L, L)] - p3)
        tpos1[pl.ds(q * PITCH, L)] = a

        @plsc.parallel_loop(0, NUM_NEG // L, step=1, unroll=2)
        def _grp(g):
            offv = negoff_v[q, pl.ds(g * L, L)]
            jg = g * L
            for i in range(L):
                on = offv[i]
                b = jnp.abs(nbuf[jg + i, pl.ds(on, L)] - p0)
                b = b + jnp.abs(nbuf[jg + i, pl.ds(on + L, L)] - p1)
                b = b + jnp.abs(nbuf[jg + i, pl.ds(on + 2 * L, L)] - p2)
                b = b + jnp.abs(nbuf[jg + i, pl.ds(on + 3 * L, L)] - p3)
                t1[pl.ds((jg + i) * PITCH, L)] = b

        for g in range(NUM_NEG // L):
            neg_out_v[q, pl.ds(g * L, L)] = (
                GAMMA - lane_reduce(t1, g * L * PITCH + ip))

    @pl.loop(0, QPW, step=2)
    def _q(q):
        wait_nbuf(nbuf0, sem_n0)
        compute(q, nbuf0)

        @pl.when(q + 2 < QPW)
        def _():
            pltpu.async_copy(emb_hbm.at[negrow_v.at[q + 2]], nbuf0, sem_n0)

        wait_nbuf(nbuf1, sem_n1)
        compute(q + 1, nbuf1)

        @pl.when(q + 3 < QPW)
        def _():
            pltpu.async_copy(emb_hbm.at[negrow_v.at[q + 3]], nbuf1, sem_n1)

    for g in range(QPW // L):
        pos_out_v[pl.ds(g * L, L)] = (
            GAMMA - lane_reduce(tpos1, g * L * PITCH + ip))

    pltpu.sync_copy(pos_out_v, pos_out_hbm.at[pl.ds(base, QPW)])
    pltpu.sync_copy(neg_out_v, neg_out_hbm.at[pl.ds(base, QPW)])


@jax.jit
def _run(x2, e2, tgtrow, tgtoff, posrow, posoff, negrow, negoff):
    mesh = plsc.VectorSubcoreMesh(core_axis_name="c", subcore_axis_name="s")
    f = pl.kernel(
        _sc_body,
        out_type=(
            jax.ShapeDtypeStruct((NUM_QUERY,), jnp.float32),
            jax.ShapeDtypeStruct((NUM_QUERY, NUM_NEG), jnp.float32),
        ),
        mesh=mesh,
        compiler_params=pltpu.CompilerParams(
            needs_layout_passes=False, use_tc_tiling_on_sc=True),
        scratch_types=[
            pltpu.VMEM((QPW,), jnp.int32),
            pltpu.VMEM((QPW,), jnp.int32),
            pltpu.VMEM((QPW,), jnp.int32),
            pltpu.VMEM((QPW,), jnp.int32),
            pltpu.VMEM((QPW, NUM_NEG), jnp.int32),
            pltpu.VMEM((QPW, NUM_NEG), jnp.int32),
            pltpu.VMEM((QPW * D,), jnp.float32),
            pltpu.VMEM((QPW * D,), jnp.float32),
            pltpu.VMEM((NUM_NEG, PAIR), jnp.float32),
            pltpu.VMEM((NUM_NEG, PAIR), jnp.float32),
            pltpu.VMEM((NUM_NEG * PITCH + L,), jnp.float32),
            pltpu.VMEM((QPW * PITCH + L,), jnp.float32),
            pltpu.VMEM((QPW,), jnp.float32),
            pltpu.VMEM((QPW, NUM_NEG), jnp.float32),
            pltpu.SemaphoreType.DMA,
            pltpu.SemaphoreType.DMA,
            pltpu.SemaphoreType.DMA,
        ],
    )
    return f(x2, e2, tgtrow, tgtoff, posrow, posoff, negrow, negoff)


def kernel(x, entity_embedding, target_node_idxes, positive_samples,
           negative_samples):
    tgt = target_node_idxes.astype(jnp.int32)
    pos = positive_samples.astype(jnp.int32)
    neg = negative_samples.astype(jnp.int32)
    x2 = x.reshape(x.shape[0] // 2, PAIR)
    t1d = _run_tr(entity_embedding.T)
    e2 = t1d.reshape(E_ROWS // 2, PAIR)
    pos_logit, neg_logit = _run(
        x2, e2,
        tgt >> 1, (tgt & 1) * D,
        pos >> 1, (pos & 1) * D,
        neg >> 1, (neg & 1) * D,
    )
    return (pos_logit[:, None], neg_logit)


# speedup vs baseline: 4.2602x; 1.0979x over previous
import jax
import jax.numpy as jnp
from jax import lax
from jax.experimental import pallas as pl
from jax.experimental.pallas import tpu as pltpu
from jax.experimental.pallas import tpu_sc as plsc

NUM_QUERY = 4096
NUM_NEG = 128
D = 64
L = 16
GAMMA = 12.0
NW = 32
QPW = NUM_QUERY // NW
PAIR = 2 * D
PITCH = L + 1

E_ROWS = 1000000
WIN = 256
NWIN = E_ROWS // WIN
TAIL = E_ROWS - NWIN * WIN
TAIL_WORKER = NWIN % NW
TP = WIN + 1


def _tr_body(et_hbm, out_hbm, tbuf0, tbuf1, obuf0, obuf1, ttail, obufp,
             semi0, semi1, semo0, semo1):
    cid = lax.axis_index("c")
    sid = lax.axis_index("s")
    wid = sid * 2 + cid
    nk = (NWIN - wid + NW - 1) // NW

    iota65 = lax.iota(jnp.int32, L) * TP

    rowsP_k = [(lax.iota(jnp.int32, L) + k * L) * TP for k in range(D // L)]

    def transpose_block(src, obuf, nent):
        @plsc.parallel_loop(0, D, step=1, unroll=4)
        def _d(d):
            for g in range(nent // L):
                obufp[pl.ds(d * TP + g * L, L)] = src[d, pl.ds(g * L, L)]

        @plsc.parallel_loop(0, nent, step=1, unroll=4)
        def _i(i):
            for k in range(D // L):
                obuf[pl.ds(i * D + k * L, L)] = plsc.load_gather(
                    obufp, [rowsP_k[k] + i])

    def ebase(k):
        c = wid + k * NW
        return pl.multiple_of(c * WIN, WIN)

    def issue_in(k, tbuf, sem):
        pltpu.async_copy(et_hbm.at[:, pl.ds(ebase(k), WIN)], tbuf, sem)

    @pl.when(nk > 0)
    def _():
        issue_in(0, tbuf0, semi0)

    @pl.when(nk > 1)
    def _():
        issue_in(1, tbuf1, semi1)

    def wait_in(tbuf, sem):
        pltpu.make_async_copy(et_hbm.at[:, pl.ds(0, WIN)], tbuf, sem).wait()

    def wait_out(obuf, sem):
        pltpu.make_async_copy(obuf, out_hbm.at[pl.ds(0, WIN * D)], sem).wait()

    def half(kk, tbuf, obuf, semi, semo):
        @pl.when(kk < nk)
        def _():
            wait_in(tbuf, semi)

            @pl.when(kk >= 2)
            def _():
                wait_out(obuf, semo)

            transpose_block(tbuf, obuf, WIN)
            pltpu.async_copy(obuf, out_hbm.at[pl.ds(ebase(kk) * D, WIN * D)],
                             semo)

            @pl.when(kk + 2 < nk)
            def _():
                issue_in(kk + 2, tbuf, semi)

    @pl.loop(0, NWIN // NW + 2, step=2)
    def _k(k):
        half(k, tbuf0, obuf0, semi0, semo0)
        half(k + 1, tbuf1, obuf1, semi1, semo1)

    @pl.when(nk > 0)
    def _():
        wait_out(obuf0, semo0)

    @pl.when(nk > 1)
    def _():
        wait_out(obuf1, semo1)

    @pl.when(wid == TAIL_WORKER)
    def _():
        pltpu.sync_copy(et_hbm.at[:, pl.ds(NWIN * WIN, TAIL)], ttail)
        transpose_block(ttail, obuf0, TAIL)
        pltpu.async_copy(obuf0.at[pl.ds(0, TAIL * D)],
                         out_hbm.at[pl.ds(NWIN * WIN * D, TAIL * D)], semo0)
        pltpu.make_async_copy(
            obuf0.at[pl.ds(0, TAIL * D)],
            out_hbm.at[pl.ds(0, TAIL * D)], semo0).wait()


@jax.jit
def _run_tr(et):
    mesh = plsc.VectorSubcoreMesh(core_axis_name="c", subcore_axis_name="s")
    f = pl.kernel(
        _tr_body,
        out_type=jax.ShapeDtypeStruct((E_ROWS * D,), jnp.float32),
        mesh=mesh,
        compiler_params=pltpu.CompilerParams(
            needs_layout_passes=False, use_tc_tiling_on_sc=True),
        scratch_types=[
            pltpu.VMEM((D, WIN), jnp.float32),
            pltpu.VMEM((D, WIN), jnp.float32),
            pltpu.VMEM((WIN * D,), jnp.float32),
            pltpu.VMEM((WIN * D,), jnp.float32),
            pltpu.VMEM((D, TAIL), jnp.float32),
            pltpu.VMEM((D * TP,), jnp.float32),
            pltpu.SemaphoreType.DMA,
            pltpu.SemaphoreType.DMA,
            pltpu.SemaphoreType.DMA,
            pltpu.SemaphoreType.DMA,
        ],
    )
    return f(et)


def _sc_body(x_hbm, emb_hbm, tgtrow_hbm, tgtoff_hbm, posrow_hbm, posoff_hbm,
             negrow_hbm, negoff_hbm,
             pos_out_hbm, neg_out_hbm,
             tgtrow_v, tgtoff_v, posrow_v, posoff_v, negrow_v, negoff_v,
             pred1, pos1, nbuf0, nbuf1, t1, tpos1, pos_out_v, neg_out_v,
             sem_a, sem_n0, sem_n1):
    cid = lax.axis_index("c")
    sid = lax.axis_index("s")
    wid = sid * 2 + cid
    base = wid * QPW

    pltpu.sync_copy(tgtrow_hbm.at[pl.ds(base, QPW)], tgtrow_v)
    pltpu.sync_copy(tgtoff_hbm.at[pl.ds(base, QPW)], tgtoff_v)
    pltpu.sync_copy(posrow_hbm.at[pl.ds(base, QPW)], posrow_v)
    pltpu.sync_copy(posoff_hbm.at[pl.ds(base, QPW)], posoff_v)
    pltpu.sync_copy(negrow_hbm.at[pl.ds(base, QPW)], negrow_v)
    pltpu.sync_copy(negoff_hbm.at[pl.ds(base, QPW)], negoff_v)

    def compact(offs_v, dst1):
        @plsc.parallel_loop(0, QPW // L, step=1)
        def _cp(g):
            offv = offs_v[pl.ds(g * L, L)]
            for i in range(L):
                q = g * L + i
                off = offv[i]
                for k in range(4):
                    dst1[pl.ds(q * D + k * L, L)] = (
                        nbuf0[q, pl.ds(off + k * L, L)])

    pltpu.async_copy(x_hbm.at[tgtrow_v], nbuf0, sem_a).wait()
    compact(tgtoff_v, pred1)
    pltpu.async_copy(emb_hbm.at[posrow_v], nbuf0, sem_a).wait()
    compact(posoff_v, pos1)

    pltpu.async_copy(emb_hbm.at[negrow_v.at[0]], nbuf0, sem_n0)
    pltpu.async_copy(emb_hbm.at[negrow_v.at[1]], nbuf1, sem_n1)

    def wait_nbuf(nbuf, sem):
        pltpu.make_async_copy(emb_hbm.at[pl.ds(0, NUM_NEG)], nbuf, sem).wait()

    iota = lax.iota(jnp.int32, L)
    ip = iota * PITCH

    def lane_reduce(tref, rows):
        vs = [plsc.load_gather(tref, [rows + c]) for c in range(L)]
        while len(vs) > 1:
            vs = [a + b for a, b in zip(vs[::2], vs[1::2])]
        return vs[0]

    def compute(q, nbuf):
        qd = q * D
        p0 = pred1[pl.ds(qd, L)]
        p1 = pred1[pl.ds(qd + L, L)]
        p2 = pred1[pl.ds(qd + 2 * L, L)]
        p3 = pred1[pl.ds(qd + 3 * L, L)]

        a = jnp.abs(pos1[pl.ds(qd, L)] - p0)
        a = a + jnp.abs(pos1[pl.ds(qd + L, L)] - p1)
        a = a + jnp.abs(pos1[pl.ds(qd + 2 * L, L)] - p2)
        a = a + jnp.abs(pos1[pl.ds(qd + 3 * L, L)] - p3)
        tpos1[pl.ds(q * PITCH, L)] = a

        @plsc.parallel_loop(0, NUM_NEG // L, step=1, unroll=2)
        def _grp(g):
            offv = negoff_v[q, pl.ds(g * L, L)]
            jg = g * L
            for i in range(L):
                on = offv[i]
                b = jnp.abs(nbuf[jg + i, pl.ds(on, L)] - p0)
                b = b + jnp.abs(nbuf[jg + i, pl.ds(on + L, L)] - p1)
                b = b + jnp.abs(nbuf[jg + i, pl.ds(on + 2 * L, L)] - p2)
                b = b + jnp.abs(nbuf[jg + i, pl.ds(on + 3 * L, L)] - p3)
                t1[pl.ds((jg + i) * PITCH, L)] = b

        for g in range(NUM_NEG // L):
            neg_out_v[q, pl.ds(g * L, L)] = (
                GAMMA - lane_reduce(t1, g * L * PITCH + ip))

    @pl.loop(0, QPW, step=2)
    def _q(q):
        wait_nbuf(nbuf0, sem_n0)
        compute(q, nbuf0)

        @pl.when(q + 2 < QPW)
        def _():
            pltpu.async_copy(emb_hbm.at[negrow_v.at[q + 2]], nbuf0, sem_n0)

        wait_nbuf(nbuf1, sem_n1)
        compute(q + 1, nbuf1)

        @pl.when(q + 3 < QPW)
        def _():
            pltpu.async_copy(emb_hbm.at[negrow_v.at[q + 3]], nbuf1, sem_n1)

    for g in range(QPW // L):
        pos_out_v[pl.ds(g * L, L)] = (
            GAMMA - lane_reduce(tpos1, g * L * PITCH + ip))

    pltpu.sync_copy(pos_out_v, pos_out_hbm.at[pl.ds(base, QPW)])
    pltpu.sync_copy(neg_out_v, neg_out_hbm.at[pl.ds(base, QPW)])


@jax.jit
def _run(x2, e2, tgtrow, tgtoff, posrow, posoff, negrow, negoff):
    mesh = plsc.VectorSubcoreMesh(core_axis_name="c", subcore_axis_name="s")
    f = pl.kernel(
        _sc_body,
        out_type=(
            jax.ShapeDtypeStruct((NUM_QUERY,), jnp.float32),
            jax.ShapeDtypeStruct((NUM_QUERY, NUM_NEG), jnp.float32),
        ),
        mesh=mesh,
        compiler_params=pltpu.CompilerParams(
            needs_layout_passes=False, use_tc_tiling_on_sc=True),
        scratch_types=[
            pltpu.VMEM((QPW,), jnp.int32),
            pltpu.VMEM((QPW,), jnp.int32),
            pltpu.VMEM((QPW,), jnp.int32),
            pltpu.VMEM((QPW,), jnp.int32),
            pltpu.VMEM((QPW, NUM_NEG), jnp.int32),
            pltpu.VMEM((QPW, NUM_NEG), jnp.int32),
            pltpu.VMEM((QPW * D,), jnp.float32),
            pltpu.VMEM((QPW * D,), jnp.float32),
            pltpu.VMEM((NUM_NEG, PAIR), jnp.float32),
            pltpu.VMEM((NUM_NEG, PAIR), jnp.float32),
            pltpu.VMEM((NUM_NEG * PITCH + L,), jnp.float32),
            pltpu.VMEM((QPW * PITCH + L,), jnp.float32),
            pltpu.VMEM((QPW,), jnp.float32),
            pltpu.VMEM((QPW, NUM_NEG), jnp.float32),
            pltpu.SemaphoreType.DMA,
            pltpu.SemaphoreType.DMA,
            pltpu.SemaphoreType.DMA,
        ],
    )
    return f(x2, e2, tgtrow, tgtoff, posrow, posoff, negrow, negoff)


def kernel(x, entity_embedding, target_node_idxes, positive_samples,
           negative_samples):
    tgt = target_node_idxes.astype(jnp.int32)
    pos = positive_samples.astype(jnp.int32)
    neg = negative_samples.astype(jnp.int32)
    x2 = x.reshape(x.shape[0] // 2, PAIR)
    t1d = _run_tr(entity_embedding.T)
    e2 = t1d.reshape(E_ROWS // 2, PAIR)
    pos_logit, neg_logit = _run(
        x2, e2,
        tgt >> 1, (tgt & 1) * D,
        pos >> 1, (pos & 1) * D,
        neg >> 1, (neg & 1) * D,
    )
    return (pos_logit[:, None], neg_logit)


# speedup vs baseline: 4.3662x; 1.0249x over previous
import jax
import jax.numpy as jnp
from jax import lax
from jax.experimental import pallas as pl
from jax.experimental.pallas import tpu as pltpu
from jax.experimental.pallas import tpu_sc as plsc

NUM_QUERY = 4096
NUM_NEG = 128
D = 64
L = 16
GAMMA = 12.0
NW = 32
QPW = NUM_QUERY // NW
PAIR = 2 * D
PITCH = L + 1

E_ROWS = 1000000
WIN = 256
NWIN = E_ROWS // WIN
TAIL = E_ROWS - NWIN * WIN
TAIL_WORKER = NWIN % NW
TP = WIN + 1


def _tr_body(et_hbm, out_hbm, tbuf0, tbuf1, obuf0, obuf1, ttail, obufp,
             semi0, semi1, semo0, semo1):
    cid = lax.axis_index("c")
    sid = lax.axis_index("s")
    wid = sid * 2 + cid
    nk = (NWIN - wid + NW - 1) // NW

    rowsP_k = [(lax.iota(jnp.int32, L) + k * L) * TP for k in range(D // L)]

    def transpose_block(src, obuf, nent):
        @plsc.parallel_loop(0, D, step=1, unroll=4)
        def _d(d):
            for g in range(nent // L):
                obufp[pl.ds(d * TP + g * L, L)] = src[d, pl.ds(g * L, L)]

        @plsc.parallel_loop(0, nent, step=1, unroll=4)
        def _i(i):
            for k in range(D // L):
                obuf[pl.ds(i * D + k * L, L)] = plsc.load_gather(
                    obufp, [rowsP_k[k] + i])

    def ebase(k):
        c = wid + k * NW
        return pl.multiple_of(c * WIN, WIN)

    def issue_in(k, tbuf, sem):
        pltpu.async_copy(et_hbm.at[:, pl.ds(ebase(k), WIN)], tbuf, sem)

    @pl.when(nk > 0)
    def _():
        issue_in(0, tbuf0, semi0)

    @pl.when(nk > 1)
    def _():
        issue_in(1, tbuf1, semi1)

    def wait_in(tbuf, sem):
        pltpu.make_async_copy(et_hbm.at[:, pl.ds(0, WIN)], tbuf, sem).wait()

    def wait_out(obuf, sem):
        pltpu.make_async_copy(obuf, out_hbm.at[pl.ds(0, WIN * D)], sem).wait()

    def half(kk, tbuf, obuf, semi, semo):
        @pl.when(kk < nk)
        def _():
            wait_in(tbuf, semi)

            @pl.when(kk >= 2)
            def _():
                wait_out(obuf, semo)

            transpose_block(tbuf, obuf, WIN)
            pltpu.async_copy(obuf, out_hbm.at[pl.ds(ebase(kk) * D, WIN * D)],
                             semo)

            @pl.when(kk + 2 < nk)
            def _():
                issue_in(kk + 2, tbuf, semi)

    @pl.loop(0, NWIN // NW + 2, step=2)
    def _k(k):
        half(k, tbuf0, obuf0, semi0, semo0)
        half(k + 1, tbuf1, obuf1, semi1, semo1)

    @pl.when(nk > 0)
    def _():
        wait_out(obuf0, semo0)

    @pl.when(nk > 1)
    def _():
        wait_out(obuf1, semo1)

    @pl.when(wid == TAIL_WORKER)
    def _():
        pltpu.sync_copy(et_hbm.at[:, pl.ds(NWIN * WIN, TAIL)], ttail)
        transpose_block(ttail, obuf0, TAIL)
        pltpu.async_copy(obuf0.at[pl.ds(0, TAIL * D)],
                         out_hbm.at[pl.ds(NWIN * WIN * D, TAIL * D)], semo0)
        pltpu.make_async_copy(
            obuf0.at[pl.ds(0, TAIL * D)],
            out_hbm.at[pl.ds(0, TAIL * D)], semo0).wait()


@jax.jit
def _run_tr(et):
    mesh = plsc.VectorSubcoreMesh(core_axis_name="c", subcore_axis_name="s")
    f = pl.kernel(
        _tr_body,
        out_type=jax.ShapeDtypeStruct((E_ROWS * D,), jnp.float32),
        mesh=mesh,
        compiler_params=pltpu.CompilerParams(
            needs_layout_passes=False, use_tc_tiling_on_sc=True),
        scratch_types=[
            pltpu.VMEM((D, WIN), jnp.float32),
            pltpu.VMEM((D, WIN), jnp.float32),
            pltpu.VMEM((WIN * D,), jnp.float32),
            pltpu.VMEM((WIN * D,), jnp.float32),
            pltpu.VMEM((D, TAIL), jnp.float32),
            pltpu.VMEM((D * TP,), jnp.float32),
            pltpu.SemaphoreType.DMA,
            pltpu.SemaphoreType.DMA,
            pltpu.SemaphoreType.DMA,
            pltpu.SemaphoreType.DMA,
        ],
    )
    return f(et)


def _sc_body(x_hbm, emb_hbm, tgtrow_hbm, tgtoff_hbm, posrow_hbm, posoff_hbm,
             negrow_hbm, negoff_hbm,
             pos_out_hbm, neg_out_hbm,
             tgtrow_v, tgtoff_v, posrow_v, posoff_v, negrow_v, negoff_v,
             pred1, pos1, nbuf0, nbuf1, t1, tpos1, pos_out_v, neg_out_v,
             sem_a, sem_n0, sem_n1):
    cid = lax.axis_index("c")
    sid = lax.axis_index("s")
    wid = sid * 2 + cid
    base = wid * QPW

    pltpu.sync_copy(tgtrow_hbm.at[pl.ds(base, QPW)], tgtrow_v)
    pltpu.sync_copy(tgtoff_hbm.at[pl.ds(base, QPW)], tgtoff_v)
    pltpu.sync_copy(posrow_hbm.at[pl.ds(base, QPW)], posrow_v)
    pltpu.sync_copy(posoff_hbm.at[pl.ds(base, QPW)], posoff_v)
    pltpu.sync_copy(negrow_hbm.at[pl.ds(base, QPW)], negrow_v)
    pltpu.sync_copy(negoff_hbm.at[pl.ds(base, QPW)], negoff_v)

    def compact(offs_v, dst1):
        @plsc.parallel_loop(0, QPW // L, step=1)
        def _cp(g):
            offv = offs_v[pl.ds(g * L, L)]
            for i in range(L):
                q = g * L + i
                off = offv[i]
                for k in range(4):
                    dst1[pl.ds(q * D + k * L, L)] = (
                        nbuf0[q, pl.ds(off + k * L, L)])

    pltpu.async_copy(x_hbm.at[tgtrow_v], nbuf0, sem_a).wait()
    compact(tgtoff_v, pred1)
    pltpu.async_copy(emb_hbm.at[posrow_v], nbuf0, sem_a).wait()
    compact(posoff_v, pos1)

    pltpu.async_copy(emb_hbm.at[negrow_v.at[0]], nbuf0, sem_n0)
    pltpu.async_copy(emb_hbm.at[negrow_v.at[1]], nbuf1, sem_n1)

    def wait_nbuf(nbuf, sem):
        pltpu.make_async_copy(emb_hbm.at[pl.ds(0, NUM_NEG)], nbuf, sem).wait()

    iota = lax.iota(jnp.int32, L)
    ip = iota * PITCH

    def lane_reduce(tref, rows):
        vs = [plsc.load_gather(tref, [rows + c]) for c in range(L)]
        while len(vs) > 1:
            vs = [a + b for a, b in zip(vs[::2], vs[1::2])]
        return vs[0]

    def compute(q, nbuf):
        qd = q * D
        p0 = pred1[pl.ds(qd, L)]
        p1 = pred1[pl.ds(qd + L, L)]
        p2 = pred1[pl.ds(qd + 2 * L, L)]
        p3 = pred1[pl.ds(qd + 3 * L, L)]

        a = jnp.abs(pos1[pl.ds(qd, L)] - p0)
        a = a + jnp.abs(pos1[pl.ds(qd + L, L)] - p1)
        a = a + jnp.abs(pos1[pl.ds(qd + 2 * L, L)] - p2)
        a = a + jnp.abs(pos1[pl.ds(qd + 3 * L, L)] - p3)
        tpos1[pl.ds(q * PITCH, L)] = a

        @plsc.parallel_loop(0, NUM_NEG // L, step=1, unroll=2)
        def _grp(g):
            offv = negoff_v[q, pl.ds(g * L, L)]
            jg = g * L
            for i in range(L):
                on = offv[i]
                b = jnp.abs(nbuf[jg + i, pl.ds(on, L)] - p0)
                b = b + jnp.abs(nbuf[jg + i, pl.ds(on + L, L)] - p1)
                b = b + jnp.abs(nbuf[jg + i, pl.ds(on + 2 * L, L)] - p2)
                b = b + jnp.abs(nbuf[jg + i, pl.ds(on + 3 * L, L)] - p3)
                t1[pl.ds((jg + i) * PITCH, L)] = b
            neg_out_v[q, pl.ds(g * L, L)] = (
                GAMMA - lane_reduce(t1, g * L * PITCH + ip))

    @pl.loop(0, QPW, step=2)
    def _q(q):
        wait_nbuf(nbuf0, sem_n0)
        compute(q, nbuf0)

        @pl.when(q + 2 < QPW)
        def _():
            pltpu.async_copy(emb_hbm.at[negrow_v.at[q + 2]], nbuf0, sem_n0)

        wait_nbuf(nbuf1, sem_n1)
        compute(q + 1, nbuf1)

        @pl.when(q + 3 < QPW)
        def _():
            pltpu.async_copy(emb_hbm.at[negrow_v.at[q + 3]], nbuf1, sem_n1)

    for g in range(QPW // L):
        pos_out_v[pl.ds(g * L, L)] = (
            GAMMA - lane_reduce(tpos1, g * L * PITCH + ip))

    pltpu.sync_copy(pos_out_v, pos_out_hbm.at[pl.ds(base, QPW)])
    pltpu.sync_copy(neg_out_v, neg_out_hbm.at[pl.ds(base, QPW)])


@jax.jit
def _run(x2, e2, tgtrow, tgtoff, posrow, posoff, negrow, negoff):
    mesh = plsc.VectorSubcoreMesh(core_axis_name="c", subcore_axis_name="s")
    f = pl.kernel(
        _sc_body,
        out_type=(
            jax.ShapeDtypeStruct((NUM_QUERY,), jnp.float32),
            jax.ShapeDtypeStruct((NUM_QUERY, NUM_NEG), jnp.float32),
        ),
        mesh=mesh,
        compiler_params=pltpu.CompilerParams(
            needs_layout_passes=False, use_tc_tiling_on_sc=True),
        scratch_types=[
            pltpu.VMEM((QPW,), jnp.int32),
            pltpu.VMEM((QPW,), jnp.int32),
            pltpu.VMEM((QPW,), jnp.int32),
            pltpu.VMEM((QPW,), jnp.int32),
            pltpu.VMEM((QPW, NUM_NEG), jnp.int32),
            pltpu.VMEM((QPW, NUM_NEG), jnp.int32),
            pltpu.VMEM((QPW * D,), jnp.float32),
            pltpu.VMEM((QPW * D,), jnp.float32),
            pltpu.VMEM((NUM_NEG, PAIR), jnp.float32),
            pltpu.VMEM((NUM_NEG, PAIR), jnp.float32),
            pltpu.VMEM((NUM_NEG * PITCH + L,), jnp.float32),
            pltpu.VMEM((QPW * PITCH + L,), jnp.float32),
            pltpu.VMEM((QPW,), jnp.float32),
            pltpu.VMEM((QPW, NUM_NEG), jnp.float32),
            pltpu.SemaphoreType.DMA,
            pltpu.SemaphoreType.DMA,
            pltpu.SemaphoreType.DMA,
        ],
    )
    return f(x2, e2, tgtrow, tgtoff, posrow, posoff, negrow, negoff)


def kernel(x, entity_embedding, target_node_idxes, positive_samples,
           negative_samples):
    tgt = target_node_idxes.astype(jnp.int32)
    pos = positive_samples.astype(jnp.int32)
    neg = negative_samples.astype(jnp.int32)
    x2 = x.reshape(x.shape[0] // 2, PAIR)
    t1d = _run_tr(entity_embedding.T)
    e2 = t1d.reshape(E_ROWS // 2, PAIR)
    pos_logit, neg_logit = _run(
        x2, e2,
        tgt >> 1, (tgt & 1) * D,
        pos >> 1, (pos & 1) * D,
        neg >> 1, (neg & 1) * D,
    )
    return (pos_logit[:, None], neg_logit)
